# baseline (speedup 1.0000x reference)
import functools
import jax
import jax.numpy as jnp
from jax import lax
from jax.experimental import pallas as pl
from jax.experimental.pallas import tpu as pltpu
from jax.experimental.pallas import tpu_sc as plsc

N = 4096
DIM = 128
H = 8
NS = 8
HD = DIM // H

BQ = 256
BC = 256
GATHER_CHUNK = 128

SC_CORES = 2
SC_SUBCORES = 16


def _knn_body(p16_ref, pblk_ref, idx_ref):
    pall = p16_ref[...]
    pblk = pblk_ref[...]
    prod = lax.dot_general(pall, pblk, (((1,), (1,)), ((), ())),
                           preferred_element_type=jnp.float32)
    d2_all = jnp.sum(pall * pall, axis=1, keepdims=True)
    dist = d2_all - 2.0 * prod

    row_iota = lax.broadcasted_iota(jnp.int32, (N, BQ), 0)
    inf = jnp.float32(jnp.inf)
    for j in range(NS):
        ij = jnp.argmin(dist, axis=0).reshape(1, BQ)
        idx_ref[j:j + 1, :] = ij
        dist = jnp.where(row_iota == ij, inf, dist)


def _knn_topk(p16, half, nh):
    hoff = half * (nh // BQ)
    return pl.pallas_call(
        _knn_body,
        grid=(nh // BQ,),
        in_specs=[
            pl.BlockSpec((N, 16), lambda i: (0, 0)),
            pl.BlockSpec((BQ, 16), lambda i: (i + hoff, 0)),
        ],
        out_specs=pl.BlockSpec((NS, BQ), lambda i: (0, i)),
        out_shape=jax.ShapeDtypeStruct((NS, nh), jnp.int32),
    )(p16, p16)


def _make_sc_gather(nrows):
    nw = SC_CORES * SC_SUBCORES
    rows_per_w = nrows // nw
    nchunks = rows_per_w // GATHER_CHUNK
    mesh = plsc.VectorSubcoreMesh(core_axis_name="c", subcore_axis_name="s")

    @functools.partial(
        pl.kernel,
        mesh=mesh,
        out_type=[
            jax.ShapeDtypeStruct((nrows, DIM), jnp.float32),
            jax.ShapeDtypeStruct((nrows, DIM), jnp.float32),
        ],
        scratch_types=[
            pltpu.VMEM((2, GATHER_CHUNK), jnp.int32),
            pltpu.VMEM((2, GATHER_CHUNK, DIM), jnp.float32),
            pltpu.VMEM((2, GATHER_CHUNK, DIM), jnp.float32),
            pltpu.SemaphoreType.DMA,
            pltpu.SemaphoreType.DMA,
            pltpu.SemaphoreType.DMA,
            pltpu.SemaphoreType.DMA,
        ],
    )
    def sc_gather(x_hbm, ptile_hbm, idx_hbm, xg_hbm, xyz_hbm,
                  idx_v, xg_v, xyz_v, semg0, semg1, semw0, semw1):
        wid = lax.axis_index("s") * SC_CORES + lax.axis_index("c")
        base = wid * rows_per_w
        semg = [semg0, semg1]
        semw = [semw0, semw1]

        def start_chunk(c):
            b = c % 2
            off = base + c * GATHER_CHUNK
            pltpu.sync_copy(idx_hbm.at[pl.ds(off, GATHER_CHUNK)], idx_v.at[b])
            g1 = pltpu.async_copy(x_hbm.at[idx_v.at[b]], xg_v.at[b], semg[b])
            g2 = pltpu.async_copy(ptile_hbm.at[idx_v.at[b]], xyz_v.at[b], semg[b])
            return g1, g2

        gath = start_chunk(0)
        wb_prev = None
        for c in range(nchunks):
            b = c % 2
            off = base + c * GATHER_CHUNK
            gath[0].wait()
            gath[1].wait()
            w1 = pltpu.async_copy(xg_v.at[b], xg_hbm.at[pl.ds(off, GATHER_CHUNK)],
                                  semw[b])
            w2 = pltpu.async_copy(xyz_v.at[b], xyz_hbm.at[pl.ds(off, GATHER_CHUNK)],
                                  semw[b])
            if c + 1 < nchunks:
                if wb_prev is not None:
                    wb_prev[0].wait()
                    wb_prev[1].wait()
                gath = start_chunk(c + 1)
            wb_prev = (w1, w2)
        wb_prev[0].wait()
        wb_prev[1].wait()

    return sc_gather


_SC_GATHER_CACHE = {}


def _sc_gather(x, ptile, idx_flat):
    nrows = idx_flat.shape[0]
    if nrows not in _SC_GATHER_CACHE:
        _SC_GATHER_CACHE[nrows] = _make_sc_gather(nrows)
    return _SC_GATHER_CACHE[nrows](x, ptile, idx_flat)


def _fused_body(xg3_ref, x0_ref, xyz3_ref, wqkv_ref,
                q_w1_ref, q_b1_ref, q_g_ref, q_be_ref, q_w2_ref, q_b2_ref,
                k_w1_ref, k_b1_ref, k_g_ref, k_be_ref, k_w2_ref, k_b2_ref,
                v_w1_ref, v_b1_ref, v_g_ref, v_be_ref, v_w2_ref, v_b2_ref,
                posw_ref, posb_ref, ppw_ref, ppb_ref,
                projw_ref, projb_ref, out_ref):
    R = NS * BC
    xg3 = xg3_ref[...]
    x0 = x0_ref[...]
    xyz3 = xyz3_ref[..., :16]
    wqkv = wqkv_ref[...]

    sr = lax.broadcasted_iota(jnp.int32, (DIM, NS * 3), 0)
    sc = lax.broadcasted_iota(jnp.int32, (DIM, NS * 3), 1)
    smat = (((sr // 16) == (sc // 3)) & ((sr % 16) == (sc % 3))).astype(jnp.float32)
    rr = lax.broadcasted_iota(jnp.int32, (16, NS * 3), 0)
    rc = lax.broadcasted_iota(jnp.int32, (16, NS * 3), 1)
    rrep = ((rr == (rc % 3)) & (rr < 3)).astype(jnp.float32)
    hr = lax.broadcasted_iota(jnp.int32, (DIM, H), 0)
    hc = lax.broadcasted_iota(jnp.int32, (DIM, H), 1)
    hs = ((hr // HD) == hc).astype(jnp.float32)
    er = lax.broadcasted_iota(jnp.int32, (H, DIM), 0)
    ec = lax.broadcasted_iota(jnp.int32, (H, DIM), 1)
    he = (er == (ec // HD)).astype(jnp.float32)
    ginv = jnp.float32(1.0) / jnp.sqrt(jnp.float32(1.0 + 1e-5))

    qkv = jnp.dot(xg3.reshape(R, DIM), wqkv,
                  preferred_element_type=jnp.float32)
    kk = qkv[:, DIM:2 * DIM]
    vv = qkv[:, 2 * DIM:]
    qkv0 = jnp.dot(x0, wqkv, preferred_element_type=jnp.float32)
    q0 = qkv0[:, :DIM]
    k0 = qkv0[:, DIM:2 * DIM]

    xyz2 = jnp.concatenate([xyz3[s] for s in range(NS)], axis=1)
    xyzrep = jnp.dot(xyz3.reshape(R, 16), rrep,
                     preferred_element_type=jnp.float32)

    def rep8(t):
        return jnp.broadcast_to(t[None, :, :], (NS,) + t.shape).reshape(R, t.shape[1])

    def branch(w1_ref, b1_ref, g_ref, be_ref, w2_ref, b2_ref):
        w1 = w1_ref[...]
        w1p = jnp.dot(smat, w1, preferred_element_type=jnp.float32)
        t1 = jnp.dot(xyzrep, w1, preferred_element_type=jnp.float32)
        t2 = jnp.dot(xyz2, w1p, preferred_element_type=jnp.float32)
        t = (t1 - rep8(t2) + b1_ref[...]) * (g_ref[...] * ginv) + be_ref[...]
        t = jnp.maximum(t, 0.0)
        return jnp.dot(t, w2_ref[...], preferred_element_type=jnp.float32) + b2_ref[...]

    pos_q = branch(q_w1_ref, q_b1_ref, q_g_ref, q_be_ref, q_w2_ref, q_b2_ref)
    pos_k = branch(k_w1_ref, k_b1_ref, k_g_ref, k_be_ref, k_w2_ref, k_b2_ref)
    pos_v = branch(v_w1_ref, v_b1_ref, v_g_ref, v_be_ref, v_w2_ref, v_b2_ref)

    gx0 = rep8(xyz3[0]) - xyz3.reshape(R, 16)
    pe0 = jnp.dot(gx0, posw_ref[...], preferred_element_type=jnp.float32) + posb_ref[...]
    p40 = jnp.dot(pe0, ppw_ref[...], preferred_element_type=jnp.float32) + ppb_ref[...]

    q0r = rep8(q0)
    k0r = rep8(k0)
    bmat = (kk + pos_q) * q0r + pos_k * k0r
    dots = jnp.dot(bmat, hs, preferred_element_type=jnp.float32)
    csum = jnp.dot(p40 * q0r, hs, preferred_element_type=jnp.float32)

    scale = jnp.float32(HD ** -0.5)
    d3 = dots.reshape(NS, BC, H) * scale
    mx = jnp.max(d3, axis=0, keepdims=True)
    e = jnp.exp(d3 - mx)
    attn = e / jnp.sum(e, axis=0, keepdims=True)
    scores = (attn + csum.reshape(NS, BC, H)) * jnp.float32(H ** -0.5)

    sexp = jnp.dot(scores.reshape(R, H), he,
                   preferred_element_type=jnp.float32)
    pmat = (vv + pos_v) * sexp
    outv = jnp.sum(pmat.reshape(NS, BC, DIM), axis=0)
    out_ref[...] = (jnp.dot(outv, projw_ref[...], preferred_element_type=jnp.float32)
                    + projb_ref[...])


def _full(shape):
    return pl.BlockSpec(shape, lambda i: tuple(0 for _ in shape))


def _fused(nh, xg, xg3, xyz3, wqkv, branch_ws, posw, posb, ppw, ppb,
           projw, projb):
    in_specs = [
        pl.BlockSpec((NS, BC, DIM), lambda i: (0, i, 0)),
        pl.BlockSpec((BC, DIM), lambda i: (i, 0)),
        pl.BlockSpec((NS, BC, DIM), lambda i: (0, i, 0)),
        _full((DIM, 3 * DIM)),
    ]
    args = [xg3, xg, xyz3, wqkv]
    for ws in branch_ws:
        in_specs += [_full((NS * 3, DIM)), _full((1, DIM)),
                     _full((1, DIM)), _full((1, DIM)), _full((DIM, DIM)),
                     _full((1, DIM))]
        args += list(ws)
    in_specs += [_full((16, DIM)), _full((1, DIM)),
                 _full((DIM, DIM)), _full((1, DIM)),
                 _full((DIM, DIM)), _full((1, DIM))]
    args += [posw, posb, ppw, ppb, projw, projb]

    return pl.pallas_call(
        _fused_body,
        grid=(nh // BC,),
        in_specs=in_specs,
        out_specs=pl.BlockSpec((BC, DIM), lambda i: (i, 0)),
        out_shape=jax.ShapeDtypeStruct((nh, DIM), jnp.float32),
    )(*args)


def kernel(p, x, Wqkv, pq_W1, pq_b1, pq_g, pq_be, pq_W2, pq_b2,
           pk_W1, pk_b1, pk_g, pk_be, pk_W2, pk_b2,
           pv_W1, pv_b1, pv_g, pv_be, pv_W2, pv_b2,
           pos_W, pos_b, pp_W, pp_b, pg_W, pg_b, proj_W, proj_b):
    f32 = jnp.float32
    p16 = jnp.pad(p.astype(f32), ((0, 0), (0, 16 - 3)))
    ptile = jnp.tile(p16, (1, NS))
    xf = x.astype(f32)

    def prep(W1, b1, g, be, W2, b2):
        return (W1, b1.reshape(1, DIM), g.reshape(1, DIM),
                be.reshape(1, DIM), W2, b2.reshape(1, DIM))

    branch_ws = [prep(pq_W1, pq_b1, pq_g, pq_be, pq_W2, pq_b2),
                 prep(pk_W1, pk_b1, pk_g, pk_be, pk_W2, pk_b2),
                 prep(pv_W1, pv_b1, pv_g, pv_be, pv_W2, pv_b2)]

    posw16 = jnp.pad(pos_W.astype(f32), ((0, 16 - 3), (0, 0)))

    nsplit = 2
    nh = N // nsplit

    def run_fused(xg, xyzg):
        return _fused(nh, xg, xg.reshape(NS, nh, DIM),
                      xyzg.reshape(NS, nh, DIM), Wqkv, branch_ws,
                      posw16, pos_b.reshape(1, DIM), pp_W,
                      pp_b.reshape(1, DIM), proj_W,
                      proj_b.reshape(1, DIM))

    idx_a = _knn_topk(p16, 0, nh).reshape(nh * NS)
    xg_a, xyz_a = _sc_gather(xf, ptile, idx_a)
    idx_b = _knn_topk(p16, 1, nh).reshape(nh * NS)
    out_a = run_fused(xg_a, xyz_a)
    xg_b, xyz_b = _sc_gather(xf, ptile, idx_b)
    out_b = run_fused(xg_b, xyz_b)
    return jnp.concatenate([out_a, out_b], axis=0)


# speedup vs baseline: 1.0268x; 1.0268x over previous
import functools
import jax
import jax.numpy as jnp
from jax import lax
from jax.experimental import pallas as pl
from jax.experimental.pallas import tpu as pltpu
from jax.experimental.pallas import tpu_sc as plsc

N = 4096
DIM = 128
H = 8
NS = 8
HD = DIM // H

BQ = 256
BC = 512
GATHER_CHUNK = 128

SC_CORES = 2
SC_SUBCORES = 16


def _knn_body(p16_ref, pblk_ref, idx_ref):
    pall = p16_ref[...]
    pblk = pblk_ref[...]
    prod = lax.dot_general(pall, pblk, (((1,), (1,)), ((), ())),
                           preferred_element_type=jnp.float32)
    d2_all = jnp.sum(pall * pall, axis=1, keepdims=True)
    dist = d2_all - 2.0 * prod

    row_iota = lax.broadcasted_iota(jnp.int32, (N, BQ), 0)
    inf = jnp.float32(jnp.inf)
    for j in range(NS):
        ij = jnp.argmin(dist, axis=0).reshape(1, BQ)
        idx_ref[j:j + 1, :] = ij
        dist = jnp.where(row_iota == ij, inf, dist)


def _knn_topk(p16, half, nh):
    hoff = half * (nh // BQ)
    return pl.pallas_call(
        _knn_body,
        grid=(nh // BQ,),
        in_specs=[
            pl.BlockSpec((N, 16), lambda i: (0, 0)),
            pl.BlockSpec((BQ, 16), lambda i: (i + hoff, 0)),
        ],
        out_specs=pl.BlockSpec((NS, BQ), lambda i: (0, i)),
        out_shape=jax.ShapeDtypeStruct((NS, nh), jnp.int32),
    )(p16, p16)


def _make_sc_gather(nrows):
    nw = SC_CORES * SC_SUBCORES
    rows_per_w = nrows // nw
    nchunks = rows_per_w // GATHER_CHUNK
    mesh = plsc.VectorSubcoreMesh(core_axis_name="c", subcore_axis_name="s")

    @functools.partial(
        pl.kernel,
        mesh=mesh,
        out_type=[
            jax.ShapeDtypeStruct((nrows, DIM), jnp.float32),
            jax.ShapeDtypeStruct((nrows, DIM), jnp.float32),
        ],
        scratch_types=[
            pltpu.VMEM((2, GATHER_CHUNK), jnp.int32),
            pltpu.VMEM((2, GATHER_CHUNK, DIM), jnp.float32),
            pltpu.VMEM((2, GATHER_CHUNK, DIM), jnp.float32),
            pltpu.SemaphoreType.DMA,
            pltpu.SemaphoreType.DMA,
            pltpu.SemaphoreType.DMA,
            pltpu.SemaphoreType.DMA,
        ],
    )
    def sc_gather(x_hbm, ptile_hbm, idx_hbm, xg_hbm, xyz_hbm,
                  idx_v, xg_v, xyz_v, semg0, semg1, semw0, semw1):
        wid = lax.axis_index("s") * SC_CORES + lax.axis_index("c")
        base = wid * rows_per_w
        semg = [semg0, semg1]
        semw = [semw0, semw1]

        def start_chunk(c):
            b = c % 2
            off = base + c * GATHER_CHUNK
            pltpu.sync_copy(idx_hbm.at[pl.ds(off, GATHER_CHUNK)], idx_v.at[b])
            g1 = pltpu.async_copy(x_hbm.at[idx_v.at[b]], xg_v.at[b], semg[b])
            g2 = pltpu.async_copy(ptile_hbm.at[idx_v.at[b]], xyz_v.at[b], semg[b])
            return g1, g2

        gath = start_chunk(0)
        wb_prev = None
        for c in range(nchunks):
            b = c % 2
            off = base + c * GATHER_CHUNK
            gath[0].wait()
            gath[1].wait()
            w1 = pltpu.async_copy(xg_v.at[b], xg_hbm.at[pl.ds(off, GATHER_CHUNK)],
                                  semw[b])
            w2 = pltpu.async_copy(xyz_v.at[b], xyz_hbm.at[pl.ds(off, GATHER_CHUNK)],
                                  semw[b])
            if c + 1 < nchunks:
                if wb_prev is not None:
                    wb_prev[0].wait()
                    wb_prev[1].wait()
                gath = start_chunk(c + 1)
            wb_prev = (w1, w2)
        wb_prev[0].wait()
        wb_prev[1].wait()

    return sc_gather


_SC_GATHER_CACHE = {}


def _sc_gather(x, ptile, idx_flat):
    nrows = idx_flat.shape[0]
    if nrows not in _SC_GATHER_CACHE:
        _SC_GATHER_CACHE[nrows] = _make_sc_gather(nrows)
    return _SC_GATHER_CACHE[nrows](x, ptile, idx_flat)


def _fused_body(xg3_ref, x0_ref, xyz3_ref, wqkv_ref,
                q_w1_ref, q_b1_ref, q_g_ref, q_be_ref, q_w2_ref, q_b2_ref,
                k_w1_ref, k_b1_ref, k_g_ref, k_be_ref, k_w2_ref, k_b2_ref,
                v_w1_ref, v_b1_ref, v_g_ref, v_be_ref, v_w2_ref, v_b2_ref,
                posw_ref, posb_ref, ppw_ref, ppb_ref,
                projw_ref, projb_ref, out_ref):
    R = NS * BC
    xg3 = xg3_ref[...]
    x0 = x0_ref[...]
    xyz3 = xyz3_ref[..., :16]
    wqkv = wqkv_ref[...]

    sr = lax.broadcasted_iota(jnp.int32, (DIM, NS * 3), 0)
    sc = lax.broadcasted_iota(jnp.int32, (DIM, NS * 3), 1)
    smat = (((sr // 16) == (sc // 3)) & ((sr % 16) == (sc % 3))).astype(jnp.float32)
    rr = lax.broadcasted_iota(jnp.int32, (16, NS * 3), 0)
    rc = lax.broadcasted_iota(jnp.int32, (16, NS * 3), 1)
    rrep = ((rr == (rc % 3)) & (rr < 3)).astype(jnp.float32)
    hr = lax.broadcasted_iota(jnp.int32, (DIM, H), 0)
    hc = lax.broadcasted_iota(jnp.int32, (DIM, H), 1)
    hs = ((hr // HD) == hc).astype(jnp.float32)
    er = lax.broadcasted_iota(jnp.int32, (H, DIM), 0)
    ec = lax.broadcasted_iota(jnp.int32, (H, DIM), 1)
    he = (er == (ec // HD)).astype(jnp.float32)
    ginv = jnp.float32(1.0) / jnp.sqrt(jnp.float32(1.0 + 1e-5))

    qkv = jnp.dot(xg3.reshape(R, DIM), wqkv,
                  preferred_element_type=jnp.float32)
    kk = qkv[:, DIM:2 * DIM]
    vv = qkv[:, 2 * DIM:]
    qkv0 = jnp.dot(x0, wqkv, preferred_element_type=jnp.float32)
    q0 = qkv0[:, :DIM]
    k0 = qkv0[:, DIM:2 * DIM]

    xyz2 = jnp.concatenate([xyz3[s] for s in range(NS)], axis=1)
    xyzrep = jnp.dot(xyz3.reshape(R, 16), rrep,
                     preferred_element_type=jnp.float32)

    def rep8(t):
        return jnp.broadcast_to(t[None, :, :], (NS,) + t.shape).reshape(R, t.shape[1])

    def branch(w1_ref, b1_ref, g_ref, be_ref, w2_ref, b2_ref):
        w1 = w1_ref[...]
        w1p = jnp.dot(smat, w1, preferred_element_type=jnp.float32)
        t1 = jnp.dot(xyzrep, w1, preferred_element_type=jnp.float32)
        t2 = jnp.dot(xyz2, w1p, preferred_element_type=jnp.float32)
        t = (t1 - rep8(t2) + b1_ref[...]) * (g_ref[...] * ginv) + be_ref[...]
        t = jnp.maximum(t, 0.0)
        return jnp.dot(t, w2_ref[...], preferred_element_type=jnp.float32) + b2_ref[...]

    pos_q = branch(q_w1_ref, q_b1_ref, q_g_ref, q_be_ref, q_w2_ref, q_b2_ref)
    pos_k = branch(k_w1_ref, k_b1_ref, k_g_ref, k_be_ref, k_w2_ref, k_b2_ref)
    pos_v = branch(v_w1_ref, v_b1_ref, v_g_ref, v_be_ref, v_w2_ref, v_b2_ref)

    gx0 = rep8(xyz3[0]) - xyz3.reshape(R, 16)
    pe0 = jnp.dot(gx0, posw_ref[...], preferred_element_type=jnp.float32) + posb_ref[...]
    p40 = jnp.dot(pe0, ppw_ref[...], preferred_element_type=jnp.float32) + ppb_ref[...]

    q0r = rep8(q0)
    k0r = rep8(k0)
    bmat = (kk + pos_q) * q0r + pos_k * k0r
    dots = jnp.dot(bmat, hs, preferred_element_type=jnp.float32)
    csum = jnp.dot(p40 * q0r, hs, preferred_element_type=jnp.float32)

    scale = jnp.float32(HD ** -0.5)
    d3 = dots.reshape(NS, BC, H) * scale
    mx = jnp.max(d3, axis=0, keepdims=True)
    e = jnp.exp(d3 - mx)
    attn = e / jnp.sum(e, axis=0, keepdims=True)
    scores = (attn + csum.reshape(NS, BC, H)) * jnp.float32(H ** -0.5)

    sexp = jnp.dot(scores.reshape(R, H), he,
                   preferred_element_type=jnp.float32)
    pmat = (vv + pos_v) * sexp
    outv = jnp.sum(pmat.reshape(NS, BC, DIM), axis=0)
    out_ref[...] = (jnp.dot(outv, projw_ref[...], preferred_element_type=jnp.float32)
                    + projb_ref[...])


def _full(shape):
    return pl.BlockSpec(shape, lambda i: tuple(0 for _ in shape))


def _fused(nh, xg, xg3, xyz3, wqkv, branch_ws, posw, posb, ppw, ppb,
           projw, projb):
    in_specs = [
        pl.BlockSpec((NS, BC, DIM), lambda i: (0, i, 0)),
        pl.BlockSpec((BC, DIM), lambda i: (i, 0)),
        pl.BlockSpec((NS, BC, DIM), lambda i: (0, i, 0)),
        _full((DIM, 3 * DIM)),
    ]
    args = [xg3, xg, xyz3, wqkv]
    for ws in branch_ws:
        in_specs += [_full((NS * 3, DIM)), _full((1, DIM)),
                     _full((1, DIM)), _full((1, DIM)), _full((DIM, DIM)),
                     _full((1, DIM))]
        args += list(ws)
    in_specs += [_full((16, DIM)), _full((1, DIM)),
                 _full((DIM, DIM)), _full((1, DIM)),
                 _full((DIM, DIM)), _full((1, DIM))]
    args += [posw, posb, ppw, ppb, projw, projb]

    return pl.pallas_call(
        _fused_body,
        grid=(nh // BC,),
        in_specs=in_specs,
        out_specs=pl.BlockSpec((BC, DIM), lambda i: (i, 0)),
        out_shape=jax.ShapeDtypeStruct((nh, DIM), jnp.float32),
    )(*args)


def kernel(p, x, Wqkv, pq_W1, pq_b1, pq_g, pq_be, pq_W2, pq_b2,
           pk_W1, pk_b1, pk_g, pk_be, pk_W2, pk_b2,
           pv_W1, pv_b1, pv_g, pv_be, pv_W2, pv_b2,
           pos_W, pos_b, pp_W, pp_b, pg_W, pg_b, proj_W, proj_b):
    f32 = jnp.float32
    p16 = jnp.pad(p.astype(f32), ((0, 0), (0, 16 - 3)))
    ptile = jnp.tile(p16, (1, NS))
    xf = x.astype(f32)

    def prep(W1, b1, g, be, W2, b2):
        return (W1, b1.reshape(1, DIM), g.reshape(1, DIM),
                be.reshape(1, DIM), W2, b2.reshape(1, DIM))

    branch_ws = [prep(pq_W1, pq_b1, pq_g, pq_be, pq_W2, pq_b2),
                 prep(pk_W1, pk_b1, pk_g, pk_be, pk_W2, pk_b2),
                 prep(pv_W1, pv_b1, pv_g, pv_be, pv_W2, pv_b2)]

    posw16 = jnp.pad(pos_W.astype(f32), ((0, 16 - 3), (0, 0)))

    nsplit = 2
    nh = N // nsplit

    def run_fused(xg, xyzg):
        return _fused(nh, xg, xg.reshape(NS, nh, DIM),
                      xyzg.reshape(NS, nh, DIM), Wqkv, branch_ws,
                      posw16, pos_b.reshape(1, DIM), pp_W,
                      pp_b.reshape(1, DIM), proj_W,
                      proj_b.reshape(1, DIM))

    idx_a = _knn_topk(p16, 0, nh).reshape(nh * NS)
    xg_a, xyz_a = _sc_gather(xf, ptile, idx_a)
    idx_b = _knn_topk(p16, 1, nh).reshape(nh * NS)
    out_a = run_fused(xg_a, xyz_a)
    xg_b, xyz_b = _sc_gather(xf, ptile, idx_b)
    out_b = run_fused(xg_b, xyz_b)
    return jnp.concatenate([out_a, out_b], axis=0)
